# baseline (speedup 1.0000x reference)
import functools

import jax
import jax.numpy as jnp
from jax import lax
from jax.experimental import pallas as pl
from jax.experimental.pallas import tpu as pltpu
from jax.experimental.pallas import tpu_sc as plsc

_BATCH = 4096
_SEQ = 50
_DIM = 64

_info = plsc.get_sparse_core_info()
_NC = _info.num_cores
_NS = _info.num_subcores
_NW = _NC * _NS
_BAT_PER_W = _BATCH // _NW
_CB = 16
_NCHUNK = _BAT_PER_W // _CB

_mesh = plsc.VectorSubcoreMesh(core_axis_name="c", subcore_axis_name="s")


@functools.partial(
    pl.kernel,
    mesh=_mesh,
    compiler_params=pltpu.CompilerParams(use_tc_tiling_on_sc=False),
    out_type=jax.ShapeDtypeStruct((_BATCH, _SEQ, _DIM), jnp.float32),
    scratch_types=[
        pltpu.VMEM((_CB, _SEQ), jnp.int32),
        pltpu.VMEM((_CB, _SEQ, _DIM), jnp.float32),
        pltpu.SemaphoreType.DMA,
    ],
)
def _gather_kernel(idx_hbm, table_hbm, out_hbm, idx_v, rows_v, sem):
    wid = lax.axis_index("s") * _NC + lax.axis_index("c")
    base = wid * _BAT_PER_W

    def body(g, carry):
        bt = base + g * _CB
        pltpu.sync_copy(idx_hbm.at[pl.ds(bt, _CB)], idx_v)
        descs = [
            pltpu.async_copy(table_hbm.at[idx_v.at[j]], rows_v.at[j], sem)
            for j in range(_CB)
        ]
        for d in descs:
            d.wait()
        pltpu.sync_copy(rows_v, out_hbm.at[pl.ds(bt, _CB)])
        return carry

    lax.fori_loop(0, _NCHUNK, body, 0)


def kernel(model_input, table):
    return _gather_kernel(model_input.astype(jnp.int32), table)


# speedup vs baseline: 1.0071x; 1.0071x over previous
import functools

import jax
import jax.numpy as jnp
from jax import lax
from jax.experimental import pallas as pl
from jax.experimental.pallas import tpu as pltpu
from jax.experimental.pallas import tpu_sc as plsc

_BATCH = 4096
_SEQ = 50
_DIM = 64
_ROWS = _BATCH * _SEQ

_info = plsc.get_sparse_core_info()
_NC = _info.num_cores
_NS = _info.num_subcores
_NW = _NC * _NS
_ROWS_PER_W = _ROWS // _NW
_CHUNK = 800
_NCHUNK = _ROWS_PER_W // _CHUNK

_mesh = plsc.VectorSubcoreMesh(core_axis_name="c", subcore_axis_name="s")


@functools.partial(
    pl.kernel,
    mesh=_mesh,
    compiler_params=pltpu.CompilerParams(use_tc_tiling_on_sc=False),
    out_type=jax.ShapeDtypeStruct((_ROWS, _DIM), jnp.float32),
    scratch_types=[
        pltpu.VMEM((2, _CHUNK), jnp.int32),
        pltpu.VMEM((2, _CHUNK, _DIM), jnp.float32),
        pltpu.SemaphoreType.DMA,
        pltpu.SemaphoreType.DMA,
        pltpu.SemaphoreType.DMA,
    ],
)
def _gather_kernel(idx_hbm, table_hbm, out_hbm, idx_v, rows_v, sem_i, sem_g, sem_w):
    wid = lax.axis_index("s") * _NC + lax.axis_index("c")
    base = wid * _ROWS_PER_W

    idx_descs = [None] * _NCHUNK
    wb_descs = [None] * _NCHUNK

    idx_descs[0] = pltpu.async_copy(
        idx_hbm.at[pl.ds(base, _CHUNK)], idx_v.at[0], sem_i
    )

    for g in range(_NCHUNK):
        s = g % 2
        start = base + g * _CHUNK
        if g + 1 < _NCHUNK:
            idx_descs[g + 1] = pltpu.async_copy(
                idx_hbm.at[pl.ds(start + _CHUNK, _CHUNK)],
                idx_v.at[(g + 1) % 2],
                sem_i,
            )
        idx_descs[g].wait()
        if g >= 2:
            wb_descs[g - 2].wait()
        gather = pltpu.async_copy(
            table_hbm.at[idx_v.at[s]], rows_v.at[s], sem_g
        )
        gather.wait()
        wb_descs[g] = pltpu.async_copy(
            rows_v.at[s], out_hbm.at[pl.ds(start, _CHUNK)], sem_w
        )

    wb_descs[_NCHUNK - 2].wait()
    wb_descs[_NCHUNK - 1].wait()


def kernel(model_input, table):
    flat_idx = model_input.astype(jnp.int32).reshape(_ROWS)
    flat_out = _gather_kernel(flat_idx, table)
    return flat_out.reshape(_BATCH, _SEQ, _DIM)


# speedup vs baseline: 1.4021x; 1.3923x over previous
import functools

import jax
import jax.numpy as jnp
from jax import lax
from jax.experimental import pallas as pl
from jax.experimental import pallas as pl
from jax.experimental.pallas import tpu as pltpu
from jax.experimental.pallas import tpu_sc as plsc

_BATCH = 4096
_SEQ = 50
_DIM = 64
_ROWS = _BATCH * _SEQ

_info = plsc.get_sparse_core_info()
_NC = _info.num_cores
_NS = _info.num_subcores
_NW = _NC * _NS
_BAT_PER_W = _BATCH // _NW
_CB = 8
_CROWS = _CB * _SEQ
_CPAD = 512
_NCHUNK = _BAT_PER_W // _CB

_mesh = plsc.VectorSubcoreMesh(core_axis_name="c", subcore_axis_name="s")


@functools.partial(
    pl.kernel,
    mesh=_mesh,
    out_type=jax.ShapeDtypeStruct((_BATCH, _SEQ, _DIM), jnp.float32),
    scratch_types=[
        pltpu.VMEM((_CPAD,), jnp.int32),
        pltpu.VMEM((_CB, _SEQ, _DIM), jnp.float32),
        pltpu.SemaphoreType.DMA,
        pltpu.SemaphoreType.DMA,
    ],
)
def _gather_kernel(idx_hbm, table_hbm, out_hbm, idx_v, rows_v, sem_g, sem_w):
    wid = lax.axis_index("s") * _NC + lax.axis_index("c")
    base_b = wid * _BAT_PER_W

    def chunk(g, carry):
        b0 = base_b + g * _CB
        r0 = b0 * _SEQ
        pltpu.sync_copy(idx_hbm.at[pl.ds(r0, _CPAD)], idx_v)

        descs = []
        for v in range(_CROWS // 16):
            vec = idx_v[pl.ds(v * 16, 16)]
            for k in range(16):
                r = v * 16 + k
                descs.append(
                    pltpu.async_copy(
                        table_hbm.at[vec[k]], rows_v.at[r // _SEQ, r % _SEQ], sem_g
                    )
                )
        for d in descs:
            d.wait()
        for b in range(_CB):
            pltpu.sync_copy(rows_v.at[b], out_hbm.at[b0 + b])
        return carry

    lax.fori_loop(0, _NCHUNK, chunk, 0)


def kernel(model_input, table):
    flat_idx = model_input.astype(jnp.int32).reshape(_ROWS)
    flat_idx = jnp.pad(flat_idx, (0, _CPAD - _CROWS))
    return _gather_kernel(flat_idx, table)


# speedup vs baseline: 1.4051x; 1.0021x over previous
import functools

import jax
import jax.numpy as jnp
from jax import lax
from jax.experimental import pallas as pl
from jax.experimental import pallas as pl
from jax.experimental.pallas import tpu as pltpu
from jax.experimental.pallas import tpu_sc as plsc

_BATCH = 4096
_SEQ = 50
_DIM = 64
_ROWS = _BATCH * _SEQ

_info = plsc.get_sparse_core_info()
_NC = _info.num_cores
_NS = _info.num_subcores
_NW = _NC * _NS
_BAT_PER_W = _BATCH // _NW
_CB = 8
_CROWS = _CB * _SEQ
_CPAD = 512
_NCHUNK = _BAT_PER_W // _CB

_mesh = plsc.VectorSubcoreMesh(core_axis_name="c", subcore_axis_name="s")


@functools.partial(
    pl.kernel,
    mesh=_mesh,
    out_type=jax.ShapeDtypeStruct((_BATCH, _SEQ, _DIM), jnp.float32),
    scratch_types=[
        pltpu.VMEM((_CPAD,), jnp.int32),
        pltpu.VMEM((_CB, _SEQ, _DIM), jnp.float32),
        pltpu.SemaphoreType.DMA,
        pltpu.SemaphoreType.DMA,
        pltpu.SemaphoreType.DMA,
        pltpu.SemaphoreType.DMA,
    ],
)
def _gather_kernel(idx_hbm, table_hbm, out_hbm, idx_v, rows_v, sem_g0, sem_g1, sem_g2, sem_g3):
    wid = lax.axis_index("s") * _NC + lax.axis_index("c")
    base_b = wid * _BAT_PER_W

    def chunk(g, carry):
        b0 = base_b + g * _CB
        r0 = b0 * _SEQ
        pltpu.sync_copy(idx_hbm.at[pl.ds(r0, _CPAD)], idx_v)

        sems = (sem_g0, sem_g1, sem_g2, sem_g3)
        descs = []
        for v in range(_CROWS // 16):
            vec = idx_v[pl.ds(v * 16, 16)]
            for k in range(16):
                r = v * 16 + k
                descs.append(
                    pltpu.async_copy(
                        table_hbm.at[vec[k]],
                        rows_v.at[r // _SEQ, r % _SEQ],
                        sems[r % 4],
                    )
                )
        for d in descs:
            d.wait()
        for b in range(_CB):
            pltpu.sync_copy(rows_v.at[b], out_hbm.at[b0 + b])
        return carry

    lax.fori_loop(0, _NCHUNK, chunk, 0)


def kernel(model_input, table):
    flat_idx = model_input.astype(jnp.int32).reshape(_ROWS)
    flat_idx = jnp.pad(flat_idx, (0, _CPAD - _CROWS))
    return _gather_kernel(flat_idx, table)
